# Initial kernel scaffold; baseline (speedup 1.0000x reference)
import jax
import jax.numpy as jnp
from jax.experimental import pallas as pl


def kernel(seq, adj, W, bias, gamma, beta):
    raise NotImplementedError("write your pallas kernel here")


# speedup vs baseline: 1.0757x; 1.0757x over previous
import jax
import jax.numpy as jnp
from jax.experimental import pallas as pl
from jax.experimental.pallas import tpu as pltpu

_N = 10000
_F = 128
_TM = 200
_NI = _N // _TM
_KC = 1024
_PC = 2000
_EPS = 1e-5


def _k_chunks():
    chunks = []
    c = 0
    while c < _N:
        chunks.append((c, min(_KC, _N - c)))
        c += _KC
    return chunks


def _body(seq_ref, adj_ref, w_ref, bias_ref, gamma_ref, beta_ref,
          out_ref, sf_ref):
    i = pl.program_id(0)

    @pl.when(i == 0)
    def _project():
        w = w_ref[...]
        for c in range(0, _N, _PC):
            s = seq_ref[pl.ds(c, _PC), :]
            sf_ref[pl.ds(c, _PC), :] = jax.lax.dot_general(
                s, w, (((1,), (1,)), ((), ())),
                preferred_element_type=jnp.float32)

    acc = jnp.zeros((_TM, _F), dtype=jnp.float32)
    for (c, wdt) in _k_chunks():
        a = adj_ref[:, pl.ds(c, wdt)]
        b = sf_ref[pl.ds(c, wdt), :]
        acc = acc + jax.lax.dot_general(
            a, b, (((1,), (0,)), ((), ())),
            preferred_element_type=jnp.float32)
    out_ref[pl.ds(i * _TM, _TM), :] = acc

    @pl.when(i == _NI - 1)
    def _finalize():
        tot = jnp.zeros((1, _F), jnp.float32)
        for c in range(0, _N, _PC):
            tot = tot + jnp.sum(out_ref[pl.ds(c, _PC), :], axis=0,
                                keepdims=True)
        mean = tot / _N
        vtot = jnp.zeros((1, _F), jnp.float32)
        for c in range(0, _N, _PC):
            d = out_ref[pl.ds(c, _PC), :] - mean
            vtot = vtot + jnp.sum(d * d, axis=0, keepdims=True)
        var = vtot / _N
        scale = gamma_ref[...] * jax.lax.rsqrt(var + _EPS)
        offset = beta_ref[...] + bias_ref[...] - mean * scale
        for c in range(0, _N, _PC):
            out_ref[pl.ds(c, _PC), :] = jnp.tanh(
                out_ref[pl.ds(c, _PC), :] * scale + offset)


def kernel(seq, adj, W, bias, gamma, beta):
    bias2 = bias.reshape(1, _F)
    gamma2 = gamma.reshape(1, _F)
    beta2 = beta.reshape(1, _F)
    return pl.pallas_call(
        _body,
        grid=(_NI,),
        in_specs=[
            pl.BlockSpec((_N, _F), lambda i: (0, 0)),
            pl.BlockSpec((_TM, _N), lambda i: (i, 0)),
            pl.BlockSpec((_F, _F), lambda i: (0, 0)),
            pl.BlockSpec((1, _F), lambda i: (0, 0)),
            pl.BlockSpec((1, _F), lambda i: (0, 0)),
            pl.BlockSpec((1, _F), lambda i: (0, 0)),
        ],
        out_specs=pl.BlockSpec((_N, _F), lambda i: (0, 0)),
        out_shape=jax.ShapeDtypeStruct((_N, _F), jnp.float32),
        scratch_shapes=[pltpu.VMEM((_N, _F), jnp.float32)],
    )(seq, adj, W, bias2, gamma2, beta2)
